# baseline (speedup 1.0000x reference)
import functools
import math

import jax
import jax.numpy as jnp
import numpy as np
from jax import lax
from jax.experimental import pallas as pl
from jax.experimental.pallas import tpu as pltpu
from jax.experimental.pallas import tpu_sc as plsc

B, N, D, C = 16, 4096, 64, 64
NB0 = 8
NSC = B - NB0
W = 272
CH = 128
HALF = N // 2
LN2 = math.log(2.0)
LOG2PI = math.log(2.0 * math.pi)

_i = np.arange(2048, dtype=np.float64)
_T0_np = np.log(1.0 + (_i + 0.5) / 2048.0)


def _ln(x, t0v):
    bits = lax.bitcast_convert_type(x, jnp.int32)
    ef = ((bits >> 23) - 127).astype(jnp.float32)
    ti = (bits >> 12) & 2047
    l0 = plsc.load_gather(t0v, [ti])
    return ef * LN2 + l0


def _body(mean_hbm, prec_hbm, tgt_hbm, t0_hbm,
          out_mean, out_prec, out_ln,
          mbuf, pbuf, tbuf, sem, accv, prtv, t0v, om_st, op_st, ln_st, shared):
    s_ax = lax.axis_index("s")
    bl = s_ax // 2
    b = NB0 + bl
    half = s_ax % 2

    pltpu.sync_copy(t0_hbm, t0v)

    zv = jnp.zeros((16,), jnp.float32)

    def zbody(k, carry):
        accv[pl.ds(k * 16, 16)] = zv
        return carry

    lax.fori_loop(0, (C * W) // 16, zbody, 0)

    iota = lax.iota(jnp.int32, 16)
    one0 = (iota == 0).astype(jnp.float32)

    pltpu.sync_copy(tgt_hbm.at[b], tbuf)

    cval = jnp.where(half == 0, 1.0, 0.0) * (iota == iota).astype(jnp.float32)

    def issue(ck, slot):
        rstart = half * 32 + ck * 2
        pltpu.async_copy(mean_hbm.at[b, pl.ds(rstart, 2)],
                         mbuf.at[pl.ds(slot * 2, 2)], sem.at[slot])
        pltpu.async_copy(prec_hbm.at[b, pl.ds(rstart, 2)],
                         pbuf.at[pl.ds(slot * 2, 2)], sem.at[slot])

    def drain(slot):
        pltpu.make_async_copy(mean_hbm.at[0, pl.ds(0, 2)],
                              mbuf.at[pl.ds(slot * 2, 2)], sem.at[slot]).wait()
        pltpu.make_async_copy(prec_hbm.at[0, pl.ds(0, 2)],
                              pbuf.at[pl.ds(slot * 2, 2)], sem.at[slot]).wait()

    issue(0, 0)
    for ck in range(16):
        slot = ck % 2
        if ck < 15:
            issue(ck + 1, 1 - slot)
        drain(slot)
        col0 = half * 32 + ck * 2

        @plsc.parallel_loop(0, N // 16, step=1, unroll=4)
        def grp_body(g):
            tv = tbuf[pl.ds(g * 16, 16)]
            tbase = tv * W
            for dd in range(2):
                row = slot * 2 + dd
                mj = mbuf[row, pl.ds(g * 16, 16)]
                pj = pbuf[row, pl.ds(g * 16, 16)]
                pmj = pj * mj
                sqj = pmj * mj
                lnj = _ln(pj, t0v)
                bid = tbase + (col0 + dd)
                plsc.addupdate_scatter(accv, [bid], pj)
                plsc.addupdate_scatter(accv, [bid + 64], pmj)
                plsc.addupdate_scatter(accv, [bid + 128], sqj)
                plsc.addupdate_scatter(accv, [bid + 192], lnj)
            if ck == 0:
                plsc.addupdate_scatter(accv, [tbase + 256], cval)

    @pl.when(half == 1)
    def _():
        pltpu.sync_copy(accv, shared.at[s_ax // 2])

    plsc.subcore_barrier()

    @pl.when(half == 0)
    def _():
        pltpu.sync_copy(shared.at[s_ax // 2], prtv)

        def cls_body(c, ln_acc):
            row = c * W

            def ld(off):
                return (accv[pl.ds(row + off, 16)]
                        + prtv[pl.ds(row + off, 16)])

            cnt = ld(256)
            ns = jnp.maximum(cnt, 1.0)
            tot = (1.0 - ns) * (0.5 * LOG2PI * D)
            for j in range(4):
                ps = ld(j * 16)
                pms = ld(64 + j * 16)
                sq = ld(128 + j * 16)
                lp = ld(192 + j * 16)
                mean = pms / ps
                expo = 0.5 * (ps * mean * mean - sq)
                lps = _ln(ps, t0v)
                tot = tot + (0.5 * (lp - lps) + expo)
                om_st[c, pl.ds(j * 16, 16)] = mean
                op_st[c, pl.ds(j * 16, 16)] = ps
            ssum = jnp.sum(tot)
            ln_acc = jnp.where(iota == (c % 16), ssum, ln_acc)
            ln_st[pl.ds((c // 16) * 16, 16)] = ln_acc
            return ln_acc

        lax.fori_loop(0, C, cls_body, jnp.zeros((16,), jnp.float32))

        pltpu.sync_copy(om_st, out_mean.at[bl])
        pltpu.sync_copy(op_st, out_prec.at[bl])
        pltpu.sync_copy(ln_st, out_ln.at[bl])


_sc_call = pl.kernel(
    _body,
    out_type=(
        jax.ShapeDtypeStruct((NSC, C, D), jnp.float32),
        jax.ShapeDtypeStruct((NSC, C, D), jnp.float32),
        jax.ShapeDtypeStruct((NSC, C), jnp.float32),
    ),
    mesh=plsc.VectorSubcoreMesh(core_axis_name="c", subcore_axis_name="s",
                                num_cores=1),
    compiler_params=pltpu.CompilerParams(needs_layout_passes=False),
    scratch_types=[
        pltpu.VMEM((4, N), jnp.float32),
        pltpu.VMEM((4, N), jnp.float32),
        pltpu.VMEM((N,), jnp.int32),
        pltpu.SemaphoreType.DMA((2,)),
        pltpu.VMEM((C * W,), jnp.float32),
        pltpu.VMEM((C * W,), jnp.float32),
        pltpu.VMEM((2048,), jnp.float32),
        pltpu.VMEM((C, D), jnp.float32),
        pltpu.VMEM((C, D), jnp.float32),
        pltpu.VMEM((C,), jnp.float32),
        pltpu.VMEM_SHARED((8, C * W), jnp.float32),
    ],
)


def _tc_body(t_ref, m_ref, p_ref, om_ref, op_ref, oln_ref):
    m_t = m_ref[0]
    p_t = p_ref[0]
    t = t_ref[0, 0]
    pm = p_t * m_t
    sq = pm * m_t
    lg = jnp.log(p_t)
    x = jnp.concatenate([p_t, pm, sq, lg], axis=0)
    oh = (t[:, None] == lax.broadcasted_iota(jnp.int32, (N, C), 1)
          ).astype(jnp.float32)
    s = jnp.dot(x, oh, preferred_element_type=jnp.float32)
    cnt = jnp.sum(oh, axis=0)
    ps = s[0:D]
    pms = s[D:2 * D]
    sqs = s[2 * D:3 * D]
    lps = s[3 * D:4 * D]
    ns = jnp.maximum(cnt, 1.0)
    mean = pms / ps
    expo = 0.5 * (ps * mean * mean - sqs)
    lnmat = 0.5 * (lps - jnp.log(ps)) + expo
    oln = lnmat.sum(axis=0) + (1.0 - ns) * (0.5 * LOG2PI * D)
    om_ref[0] = mean
    op_ref[0] = ps
    oln_ref[0, 0, :] = oln


def tc_call(means, precisions, targets, nb):
    m_t = jnp.swapaxes(means, 1, 2)
    p_t = jnp.swapaxes(precisions, 1, 2)
    t3 = targets.reshape(B, 1, N)
    _call = pl.pallas_call(
        _tc_body,
        grid=(nb,),
        in_specs=[
            pl.BlockSpec((1, 1, N), lambda i: (i, 0, 0)),
            pl.BlockSpec((1, D, N), lambda i: (i, 0, 0)),
            pl.BlockSpec((1, D, N), lambda i: (i, 0, 0)),
        ],
        out_specs=[
            pl.BlockSpec((1, D, C), lambda i: (i, 0, 0)),
            pl.BlockSpec((1, D, C), lambda i: (i, 0, 0)),
            pl.BlockSpec((1, 1, C), lambda i: (i, 0, 0)),
        ],
        out_shape=[
            jax.ShapeDtypeStruct((nb, D, C), jnp.float32),
            jax.ShapeDtypeStruct((nb, D, C), jnp.float32),
            jax.ShapeDtypeStruct((nb, 1, C), jnp.float32),
        ],
    )
    out = _call(t3, m_t, p_t)
    return (jnp.swapaxes(out[0], 1, 2), jnp.swapaxes(out[1], 1, 2),
            out[2].reshape(nb, C))


@jax.jit
def kernel(means, precisions, targets):
    t0 = jnp.asarray(_T0_np, dtype=jnp.float32)
    m_t = jnp.swapaxes(means, 1, 2)
    p_t = jnp.swapaxes(precisions, 1, 2)
    sm, sp, sl = _sc_call(m_t, p_t, targets, t0)
    tm, tp, tl = tc_call(means, precisions, targets, NB0)
    return (jnp.concatenate([tm, sm], axis=0),
            jnp.concatenate([tp, sp], axis=0),
            jnp.concatenate([tl, sl], axis=0))


# speedup vs baseline: 2.2134x; 2.2134x over previous
import functools
import math

import jax
import jax.numpy as jnp
import numpy as np
from jax import lax
from jax.experimental import pallas as pl
from jax.experimental.pallas import tpu as pltpu
from jax.experimental.pallas import tpu_sc as plsc

B, N, D, C = 16, 4096, 64, 64
NB0 = 8
NSC = B - NB0
W = 273
CH = 128
HALF = N // 2
LN2 = math.log(2.0)
LOG2PI = math.log(2.0 * math.pi)

_i = np.arange(2048, dtype=np.float64)
_T0_np = np.log(1.0 + (_i + 0.5) / 2048.0)


def _ln(x, t0v):
    bits = lax.bitcast_convert_type(x, jnp.int32)
    ef = ((bits >> 23) - 127).astype(jnp.float32)
    ti = (bits >> 12) & 2047
    l0 = plsc.load_gather(t0v, [ti])
    return ef * LN2 + l0


def _body(mean_hbm, prec_hbm, tgt_hbm, t0_hbm,
          out_mean, out_prec, out_ln,
          mbuf, pbuf, tbuf, sem, accv, prtv, t0v, om_st, op_st, ln_st, shared):
    s_ax = lax.axis_index("s")
    bl = s_ax // 2
    b = NB0 + bl
    half = s_ax % 2

    pltpu.sync_copy(t0_hbm, t0v)

    zv = jnp.zeros((16,), jnp.float32)

    def zbody(k, carry):
        accv[pl.ds(k * 16, 16)] = zv
        return carry

    lax.fori_loop(0, (C * W) // 16, zbody, 0)

    iota = lax.iota(jnp.int32, 16)
    one0 = (iota == 0).astype(jnp.float32)

    pltpu.sync_copy(tgt_hbm.at[b], tbuf)

    cval = jnp.where(half == 0, 1.0, 0.0) * (iota == iota).astype(jnp.float32)

    def issue(ck, slot):
        rstart = half * 32 + ck * 2
        pltpu.async_copy(mean_hbm.at[b, pl.ds(rstart, 2)],
                         mbuf.at[pl.ds(slot * 2, 2)], sem.at[slot])
        pltpu.async_copy(prec_hbm.at[b, pl.ds(rstart, 2)],
                         pbuf.at[pl.ds(slot * 2, 2)], sem.at[slot])

    def drain(slot):
        pltpu.make_async_copy(mean_hbm.at[0, pl.ds(0, 2)],
                              mbuf.at[pl.ds(slot * 2, 2)], sem.at[slot]).wait()
        pltpu.make_async_copy(prec_hbm.at[0, pl.ds(0, 2)],
                              pbuf.at[pl.ds(slot * 2, 2)], sem.at[slot]).wait()

    issue(0, 0)
    for ck in range(16):
        slot = ck % 2
        if ck < 15:
            issue(ck + 1, 1 - slot)
        drain(slot)
        col0 = half * 32 + ck * 2

        @plsc.parallel_loop(0, N // 16, step=1, unroll=4)
        def grp_body(g):
            tv = tbuf[pl.ds(g * 16, 16)]
            tbase = tv * W
            for dd in range(2):
                row = slot * 2 + dd
                mj = mbuf[row, pl.ds(g * 16, 16)]
                pj = pbuf[row, pl.ds(g * 16, 16)]
                pmj = pj * mj
                sqj = pmj * mj
                lnj = _ln(pj, t0v)
                bid = tbase + (col0 + dd)
                plsc.addupdate_scatter(accv, [bid], pj)
                plsc.addupdate_scatter(accv, [bid + 64], pmj)
                plsc.addupdate_scatter(accv, [bid + 128], sqj)
                plsc.addupdate_scatter(accv, [bid + 192], lnj)
            if ck == 0:
                plsc.addupdate_scatter(accv, [tbase + 256], cval)

    @pl.when(half == 1)
    def _():
        pltpu.sync_copy(accv, shared.at[s_ax // 2])

    plsc.subcore_barrier()

    @pl.when(half == 0)
    def _():
        pltpu.sync_copy(shared.at[s_ax // 2], prtv)

        def cls_body(c, ln_acc):
            row = c * W

            def ld(off):
                return (accv[pl.ds(row + off, 16)]
                        + prtv[pl.ds(row + off, 16)])

            cnt = ld(256)
            ns = jnp.maximum(cnt, 1.0)
            tot = (1.0 - ns) * (0.5 * LOG2PI * D)
            for j in range(4):
                ps = ld(j * 16)
                pms = ld(64 + j * 16)
                sq = ld(128 + j * 16)
                lp = ld(192 + j * 16)
                mean = pms / ps
                expo = 0.5 * (ps * mean * mean - sq)
                lps = _ln(ps, t0v)
                tot = tot + (0.5 * (lp - lps) + expo)
                om_st[c, pl.ds(j * 16, 16)] = mean
                op_st[c, pl.ds(j * 16, 16)] = ps
            ssum = jnp.sum(tot)
            ln_acc = jnp.where(iota == (c % 16), ssum, ln_acc)
            ln_st[pl.ds((c // 16) * 16, 16)] = ln_acc
            return ln_acc

        lax.fori_loop(0, C, cls_body, jnp.zeros((16,), jnp.float32))

        pltpu.sync_copy(om_st, out_mean.at[bl])
        pltpu.sync_copy(op_st, out_prec.at[bl])
        pltpu.sync_copy(ln_st, out_ln.at[bl])


_sc_call = pl.kernel(
    _body,
    out_type=(
        jax.ShapeDtypeStruct((NSC, C, D), jnp.float32),
        jax.ShapeDtypeStruct((NSC, C, D), jnp.float32),
        jax.ShapeDtypeStruct((NSC, C), jnp.float32),
    ),
    mesh=plsc.VectorSubcoreMesh(core_axis_name="c", subcore_axis_name="s",
                                num_cores=1),
    compiler_params=pltpu.CompilerParams(needs_layout_passes=False),
    scratch_types=[
        pltpu.VMEM((4, N), jnp.float32),
        pltpu.VMEM((4, N), jnp.float32),
        pltpu.VMEM((N,), jnp.int32),
        pltpu.SemaphoreType.DMA((2,)),
        pltpu.VMEM((C * W,), jnp.float32),
        pltpu.VMEM((C * W,), jnp.float32),
        pltpu.VMEM((2048,), jnp.float32),
        pltpu.VMEM((C, D), jnp.float32),
        pltpu.VMEM((C, D), jnp.float32),
        pltpu.VMEM((C,), jnp.float32),
        pltpu.VMEM_SHARED((8, C * W), jnp.float32),
    ],
)


def _tc_body(t_ref, m_ref, p_ref, om_ref, op_ref, oln_ref):
    m_t = m_ref[0]
    p_t = p_ref[0]
    t = t_ref[0, 0]
    pm = p_t * m_t
    sq = pm * m_t
    lg = jnp.log(p_t)
    x = jnp.concatenate([p_t, pm, sq, lg], axis=0)
    oh = (t[:, None] == lax.broadcasted_iota(jnp.int32, (N, C), 1)
          ).astype(jnp.float32)
    s = jnp.dot(x, oh, preferred_element_type=jnp.float32)
    cnt = jnp.sum(oh, axis=0)
    ps = s[0:D]
    pms = s[D:2 * D]
    sqs = s[2 * D:3 * D]
    lps = s[3 * D:4 * D]
    ns = jnp.maximum(cnt, 1.0)
    mean = pms / ps
    expo = 0.5 * (ps * mean * mean - sqs)
    lnmat = 0.5 * (lps - jnp.log(ps)) + expo
    oln = lnmat.sum(axis=0) + (1.0 - ns) * (0.5 * LOG2PI * D)
    om_ref[0] = mean
    op_ref[0] = ps
    oln_ref[0, 0, :] = oln


def tc_call(means, precisions, targets, nb):
    m_t = jnp.swapaxes(means, 1, 2)
    p_t = jnp.swapaxes(precisions, 1, 2)
    t3 = targets.reshape(B, 1, N)
    _call = pl.pallas_call(
        _tc_body,
        grid=(nb,),
        in_specs=[
            pl.BlockSpec((1, 1, N), lambda i: (i, 0, 0)),
            pl.BlockSpec((1, D, N), lambda i: (i, 0, 0)),
            pl.BlockSpec((1, D, N), lambda i: (i, 0, 0)),
        ],
        out_specs=[
            pl.BlockSpec((1, D, C), lambda i: (i, 0, 0)),
            pl.BlockSpec((1, D, C), lambda i: (i, 0, 0)),
            pl.BlockSpec((1, 1, C), lambda i: (i, 0, 0)),
        ],
        out_shape=[
            jax.ShapeDtypeStruct((nb, D, C), jnp.float32),
            jax.ShapeDtypeStruct((nb, D, C), jnp.float32),
            jax.ShapeDtypeStruct((nb, 1, C), jnp.float32),
        ],
    )
    out = _call(t3, m_t, p_t)
    return (jnp.swapaxes(out[0], 1, 2), jnp.swapaxes(out[1], 1, 2),
            out[2].reshape(nb, C))


@jax.jit
def kernel(means, precisions, targets):
    t0 = jnp.asarray(_T0_np, dtype=jnp.float32)
    m_t = jnp.swapaxes(means, 1, 2)
    p_t = jnp.swapaxes(precisions, 1, 2)
    sm, sp, sl = _sc_call(m_t, p_t, targets, t0)
    tm, tp, tl = tc_call(means, precisions, targets, NB0)
    return (jnp.concatenate([tm, sm], axis=0),
            jnp.concatenate([tp, sp], axis=0),
            jnp.concatenate([tl, sl], axis=0))
